# baseline (speedup 1.0000x reference)
import functools

import jax
import jax.numpy as jnp
from jax import lax
from jax.experimental import pallas as pl
from jax.experimental.pallas import tpu as pltpu
from jax.experimental.pallas import tpu_sc as plsc

_K = 65536
_D = 128
_B = 16384
_NC = 2
_NS = 16
_NW = _NC * _NS
_RW = _K // _NW
_CH = 512
_NCH = _RW // _CH
_SB = 128


@functools.partial(
    pl.kernel,
    out_type=jax.ShapeDtypeStruct((_K, _D), jnp.float32),
    mesh=plsc.VectorSubcoreMesh(core_axis_name="c", subcore_axis_name="s"),
    compiler_params=pltpu.CompilerParams(needs_layout_passes=False),
    scratch_types=[
        pltpu.VMEM((_B,), jnp.int32),
        pltpu.VMEM((_RW,), jnp.int32),
        pltpu.VMEM((_RW + _SB,), jnp.int32),
        pltpu.VMEM((_RW + _SB,), jnp.int32),
        pltpu.VMEM((_CH, _D), jnp.float32),
        pltpu.VMEM((_SB, _D), jnp.float32),
        pltpu.VMEM((_SB, _D), jnp.float32),
        pltpu.SemaphoreType.DMA,
        pltpu.SemaphoreType.DMA,
        pltpu.SemaphoreType.DMA,
    ],
)
def _sc_update(mem_hbm, x_hbm, y_hbm, out_hbm,
               ys, wtab, wb, wk, slab, xb0, xb1, sem, gsem0, gsem1):
    wid = lax.axis_index("s") * _NC + lax.axis_index("c")
    lo = wid * _RW
    hi = lo + _RW
    iota = lax.iota(jnp.int32, 16)

    pltpu.async_copy(y_hbm, ys, sem).wait()

    def initw(i, carry):
        wtab[pl.ds(i * 16, 16)] = jnp.full((16,), -1, jnp.int32)
        return carry

    lax.fori_loop(0, _RW // 16, initw, 0)

    def mark(i, carry):
        kv = ys[pl.ds(i * 16, 16)]
        mk = (kv >= lo) & (kv < hi)
        _, lastm = plsc.scan_count(kv, mask=mk)
        plsc.store_scatter(wtab, [kv - lo], i * 16 + iota, mask=mk & lastm)
        return carry

    lax.fori_loop(0, _B // 16, mark, 0)

    bounds = [jnp.int32(0)]
    cnt = jnp.int32(0)
    for c in range(_NCH):
        def extract(i, cnt):
            wv = wtab[pl.ds(i * 16, 16)]
            mk = wv >= 0
            cs = plsc.cumsum(mk.astype(jnp.int32))
            pos = cnt + cs - 1
            plsc.store_scatter(wb, [pos], wv, mask=mk)
            plsc.store_scatter(wk, [pos], i * 16 + iota, mask=mk)
            return cnt + jnp.sum(mk.astype(jnp.int32))

        cnt = lax.fori_loop(c * (_CH // 16), (c + 1) * (_CH // 16), extract, cnt)
        bounds.append(cnt)

    def fire_batch(s0, end, xbuf, gsem):
        n1 = jnp.minimum(end - s0, _SB)
        ng = (n1 + 15) // 16

        def fire(g, carry):
            lanes = g * 16 + iota
            bv = plsc.load_gather(wb, [s0 + lanes])
            bv = jnp.where(lanes < n1, bv, 0)
            pltpu.async_copy(x_hbm.at[bv], xbuf.at[pl.ds(g * 16, 16)], gsem)
            return carry

        lax.fori_loop(0, ng, fire, 0)

    def process_batch(s0, end, c, xbuf, gsem):
        valid = jnp.minimum(end - s0, _SB)
        ng = (valid + 15) // 16

        def drain(g, carry):
            pltpu.make_async_copy(
                x_hbm.at[pl.ds(0, 16)],
                xbuf.at[pl.ds(g * 16, 16)], gsem).wait()
            return carry

        lax.fori_loop(0, ng, drain, 0)

        def row(i, carry):
            ckv = plsc.load_gather(
                wk, [jnp.full((16,), s0 + i, jnp.int32)]) - c * _CH
            iv = jnp.full((16,), i, jnp.int32)
            us = []
            acc = jnp.zeros((16,), jnp.float32)
            for j in range(_D // 16):
                col = j * 16 + iota
                mv = plsc.load_gather(slab, [ckv, col])
                xv = plsc.load_gather(xbuf, [iv, col])
                u = (mv + xv) * 0.5
                us.append(u)
                acc = acc + u * u
            sv = jnp.full((16,), jnp.sum(acc), jnp.float32)
            r = plsc.bitcast(
                jnp.int32(0x5F3759DF) - (plsc.bitcast(sv, jnp.int32) >> 1),
                jnp.float32)
            hx = sv * 0.5
            r = r * (1.5 - hx * r * r)
            r = r * (1.5 - hx * r * r)
            r = r * (1.5 - hx * r * r)
            r = r * (1.5 - hx * r * r)
            r = jnp.minimum(r, 1e12)
            for j in range(_D // 16):
                plsc.store_scatter(slab, [ckv, j * 16 + iota], us[j] * r)
            return carry

        lax.fori_loop(0, valid, row, 0)

    bufs = [(xb0, gsem0), (xb1, gsem1)]

    fire_batch(bounds[0], bounds[1], xb0, gsem0)
    for c in range(_NCH):
        row0 = pl.multiple_of(lo + c * _CH, _CH)
        cp_in = pltpu.async_copy(mem_hbm.at[pl.ds(row0, _CH)], slab, sem)
        if c + 1 < _NCH:
            nbuf, nsem = bufs[(c + 1) % 2]
            fire_batch(bounds[c + 1], bounds[c + 2], nbuf, nsem)
        cp_in.wait()

        start = bounds[c]
        end = bounds[c + 1]
        xbuf, gsem = bufs[c % 2]
        process_batch(start, end, c, xbuf, gsem)

        nb = (end - start + _SB - 1) // _SB

        def sub(t, carry):
            s0 = start + t * _SB
            fire_batch(s0, end, xbuf, gsem)
            process_batch(s0, end, c, xbuf, gsem)
            return carry

        lax.fori_loop(1, nb, sub, 0)
        pltpu.async_copy(slab, out_hbm.at[pl.ds(row0, _CH)], sem).wait()


def kernel(memory, x, y):
    return _sc_update(memory, x, y)


# speedup vs baseline: 1.0354x; 1.0354x over previous
import functools

import jax
import jax.numpy as jnp
from jax import lax
from jax.experimental import pallas as pl
from jax.experimental.pallas import tpu as pltpu
from jax.experimental.pallas import tpu_sc as plsc

_K = 65536
_D = 128
_B = 16384
_NC = 2
_NS = 16
_NW = _NC * _NS
_RW = _K // _NW
_CH = 512
_NCH = _RW // _CH
_SB = 128


@functools.partial(
    pl.kernel,
    out_type=jax.ShapeDtypeStruct((_K, _D), jnp.float32),
    mesh=plsc.VectorSubcoreMesh(core_axis_name="c", subcore_axis_name="s"),
    compiler_params=pltpu.CompilerParams(needs_layout_passes=False),
    scratch_types=[
        pltpu.VMEM((_B,), jnp.int32),
        pltpu.VMEM((_RW,), jnp.int32),
        pltpu.VMEM((_RW + _SB,), jnp.int32),
        pltpu.VMEM((_RW + _SB,), jnp.int32),
        pltpu.VMEM((_CH, _D), jnp.float32),
        pltpu.VMEM((_SB, _D), jnp.float32),
        pltpu.VMEM((_SB, _D), jnp.float32),
        pltpu.SemaphoreType.DMA,
        pltpu.SemaphoreType.DMA,
        pltpu.SemaphoreType.DMA,
    ],
)
def _sc_update(mem_hbm, x_hbm, y_hbm, out_hbm,
               ys, wtab, wb, wk, slab, xb0, xb1, sem, gsem0, gsem1):
    wid = lax.axis_index("s") * _NC + lax.axis_index("c")
    lo = wid * _RW
    hi = lo + _RW
    iota = lax.iota(jnp.int32, 16)

    cp_y = pltpu.async_copy(y_hbm, ys, gsem0)
    cp_slab0 = pltpu.async_copy(mem_hbm.at[pl.ds(pl.multiple_of(lo, _CH), _CH)],
                                slab, sem)
    cp_y.wait()

    def initw(i, carry):
        wtab[pl.ds(i * 16, 16)] = jnp.full((16,), -1, jnp.int32)
        return carry

    lax.fori_loop(0, _RW // 16, initw, 0)

    def mark(i, carry):
        kv0 = ys[pl.ds(i * 32, 16)]
        kv1 = ys[pl.ds(i * 32 + 16, 16)]
        mk0 = (kv0 >= lo) & (kv0 < hi)
        mk1 = (kv1 >= lo) & (kv1 < hi)
        _, lastm0 = plsc.scan_count(kv0, mask=mk0)
        _, lastm1 = plsc.scan_count(kv1, mask=mk1)
        plsc.store_scatter(wtab, [kv0 - lo], i * 32 + iota, mask=mk0 & lastm0)
        plsc.store_scatter(wtab, [kv1 - lo], i * 32 + 16 + iota,
                           mask=mk1 & lastm1)
        return carry

    lax.fori_loop(0, _B // 32, mark, 0)

    bounds = [jnp.int32(0)]
    cnt = jnp.int32(0)
    for c in range(_NCH):
        def extract(i, cnt):
            wv = wtab[pl.ds(i * 16, 16)]
            mk = wv >= 0
            cs = plsc.cumsum(mk.astype(jnp.int32))
            pos = cnt + cs - 1
            plsc.store_scatter(wb, [pos], wv, mask=mk)
            plsc.store_scatter(wk, [pos], i * 16 + iota, mask=mk)
            return cnt + jnp.sum(mk.astype(jnp.int32))

        cnt = lax.fori_loop(c * (_CH // 16), (c + 1) * (_CH // 16), extract, cnt)
        bounds.append(cnt)

    def fire_batch(s0, end, xbuf, gsem):
        n1 = jnp.minimum(end - s0, _SB)
        ng = (n1 + 15) // 16

        def fire(g, carry):
            lanes = g * 16 + iota
            bv = plsc.load_gather(wb, [s0 + lanes])
            bv = jnp.where(lanes < n1, bv, 0)
            pltpu.async_copy(x_hbm.at[bv], xbuf.at[pl.ds(g * 16, 16)], gsem)
            return carry

        lax.fori_loop(0, ng, fire, 0)

    def process_batch(s0, end, c, xbuf, gsem):
        valid = jnp.minimum(end - s0, _SB)
        ng = (valid + 15) // 16

        def drain(g, carry):
            pltpu.make_async_copy(
                x_hbm.at[pl.ds(0, 16)],
                xbuf.at[pl.ds(g * 16, 16)], gsem).wait()
            return carry

        lax.fori_loop(0, ng, drain, 0)

        def row(i, carry):
            ckv = plsc.load_gather(
                wk, [jnp.full((16,), s0 + i, jnp.int32)]) - c * _CH
            iv = jnp.full((16,), i, jnp.int32)
            us = []
            acc = jnp.zeros((16,), jnp.float32)
            for j in range(_D // 16):
                col = j * 16 + iota
                mv = plsc.load_gather(slab, [ckv, col])
                xv = plsc.load_gather(xbuf, [iv, col])
                u = (mv + xv) * 0.5
                us.append(u)
                acc = acc + u * u
            sv = jnp.full((16,), jnp.sum(acc), jnp.float32)
            r = plsc.bitcast(
                jnp.int32(0x5F3759DF) - (plsc.bitcast(sv, jnp.int32) >> 1),
                jnp.float32)
            hx = sv * 0.5
            r = r * (1.5 - hx * r * r)
            r = r * (1.5 - hx * r * r)
            r = r * (1.5 - hx * r * r)
            r = r * (1.5 - hx * r * r)
            r = jnp.minimum(r, 1e12)
            for j in range(_D // 16):
                plsc.store_scatter(slab, [ckv, j * 16 + iota], us[j] * r)
            return carry

        lax.fori_loop(0, valid, row, 0)

    bufs = [(xb0, gsem0), (xb1, gsem1)]

    fire_batch(bounds[0], bounds[1], xb0, gsem0)
    for c in range(_NCH):
        row0 = pl.multiple_of(lo + c * _CH, _CH)
        cp_in = cp_slab0 if c == 0 else pltpu.async_copy(
            mem_hbm.at[pl.ds(row0, _CH)], slab, sem)
        if c + 1 < _NCH:
            nbuf, nsem = bufs[(c + 1) % 2]
            fire_batch(bounds[c + 1], bounds[c + 2], nbuf, nsem)
        cp_in.wait()

        start = bounds[c]
        end = bounds[c + 1]
        xbuf, gsem = bufs[c % 2]
        process_batch(start, end, c, xbuf, gsem)

        nb = (end - start + _SB - 1) // _SB

        def sub(t, carry):
            s0 = start + t * _SB
            fire_batch(s0, end, xbuf, gsem)
            process_batch(s0, end, c, xbuf, gsem)
            return carry

        lax.fori_loop(1, nb, sub, 0)
        pltpu.async_copy(slab, out_hbm.at[pl.ds(row0, _CH)], sem).wait()


def kernel(memory, x, y):
    return _sc_update(memory, x, y)
